# Initial kernel scaffold; baseline (speedup 1.0000x reference)
import jax
import jax.numpy as jnp
from jax.experimental import pallas as pl


def kernel(x, pos, batch, x_skip, pos_skip, batch_skip, W1, b1, W2, b2):
    raise NotImplementedError("write your pallas kernel here")


# speedup vs baseline: 7.7302x; 7.7302x over previous
import functools

import jax
import jax.numpy as jnp
from jax import lax
from jax.experimental import pallas as pl
from jax.experimental.pallas import tpu as pltpu
from jax.experimental.pallas import tpu_sc as plsc

QB = 128
MB = 512
GCH = 128


def _rne_bf16(v):
    u = lax.bitcast_convert_type(v, jnp.uint32)
    r = (u + jnp.uint32(0x7FFF) + ((u >> jnp.uint32(16)) & jnp.uint32(1))) \
        & jnp.uint32(0xFFFF0000)
    return lax.bitcast_convert_type(r, jnp.float32)


def _knn_body(pos_t_ref, p_sq_ref, qaug_ref, idx_ref, wn_ref):
    n_c = pos_t_ref.shape[1]
    qb = qaug_ref.shape[0]
    qp = None
    for c in range(3):
        pc = _rne_bf16(pos_t_ref[c:c + 1, :])
        qc = _rne_bf16(qaug_ref[:, c:c + 1])
        qp = qc * pc if qp is None else qp + qc * pc
    q_sq = qaug_ref[:, 3:4]
    d2 = (q_sq + p_sq_ref[...]) - 2.0 * qp

    iota = lax.broadcasted_iota(jnp.int32, (qb, n_c), 1)
    inf = jnp.float32(jnp.inf)
    d = d2
    ws, ims = [], []
    wsum = None
    for t in range(3):
        m = jnp.min(d, axis=1, keepdims=True)
        im = jnp.min(jnp.where(d == m, iota, n_c), axis=1, keepdims=True)
        w = 1.0 / jnp.maximum(m, 1e-16)
        ws.append(w)
        ims.append(im)
        wsum = w if wsum is None else wsum + w
        if t < 2:
            d = jnp.where(iota == im, inf, d)
    for t in range(3):
        idx_ref[:, t:t + 1] = ims[t]
        wn_ref[:, t:t + 1] = ws[t] / wsum


def _knn(pos_t_r, p_sq_row, qaug):
    n_c = pos_t_r.shape[1]
    n_f = qaug.shape[0]
    return pl.pallas_call(
        _knn_body,
        grid=(n_f // QB,),
        in_specs=[
            pl.BlockSpec((3, n_c), lambda i: (0, 0)),
            pl.BlockSpec((1, n_c), lambda i: (0, 0)),
            pl.BlockSpec((QB, 4), lambda i: (i, 0)),
        ],
        out_specs=[
            pl.BlockSpec((QB, 3), lambda i: (i, 0)),
            pl.BlockSpec((QB, 3), lambda i: (i, 0)),
        ],
        out_shape=[
            jax.ShapeDtypeStruct((n_f, 3), jnp.int32),
            jax.ShapeDtypeStruct((n_f, 3), jnp.float32),
        ],
    )(pos_t_r, p_sq_row, qaug)


@functools.cache
def _make_gather(n_c, n_f, d_in):
    info = plsc.get_sparse_core_info()
    n_cores = info.num_cores
    nw = n_cores * info.num_subcores
    bpw = n_f // nw
    nch = bpw // GCH
    mesh = plsc.VectorSubcoreMesh(core_axis_name="c", subcore_axis_name="s")

    @functools.partial(
        pl.kernel,
        out_type=jax.ShapeDtypeStruct((3 * n_f, d_in), jnp.float32),
        mesh=mesh,
        scratch_types=[
            pltpu.VMEM((GCH,), jnp.int32),
            pltpu.VMEM((GCH, d_in), jnp.float32),
            pltpu.SemaphoreType.DMA,
        ],
    )
    def gather(x_hbm, idx_hbm, out_hbm, idx_v, rows_v, sem):
        wid = lax.axis_index("s") * n_cores + lax.axis_index("c")
        base = wid * bpw

        def chunk(i, carry):
            qoff = base + i * GCH
            for j in range(3):
                foff = j * n_f + qoff
                pltpu.sync_copy(idx_hbm.at[pl.ds(foff, GCH)], idx_v)
                pltpu.async_copy(x_hbm.at[idx_v], rows_v, sem).wait()
                pltpu.sync_copy(rows_v, out_hbm.at[pl.ds(foff, GCH)])
            return carry

        lax.fori_loop(0, nch, chunk, 0)

    return gather


def _mlp_body(f0_ref, f1_ref, f2_ref, wn_ref, xs_ref, w1a_ref, w1b_ref,
              b1_ref, w2_ref, b2_ref, out_ref):
    y = None
    for t, f_ref in enumerate((f0_ref, f1_ref, f2_ref)):
        ft = f_ref[...]
        wt = wn_ref[:, t:t + 1]
        y = wt * ft if y is None else y + wt * ft
    h = (lax.dot_general(y, w1a_ref[...], (((1,), (0,)), ((), ())),
                         preferred_element_type=jnp.float32)
         + lax.dot_general(xs_ref[...], w1b_ref[...], (((1,), (0,)), ((), ())),
                           preferred_element_type=jnp.float32)
         + b1_ref[...])
    h = jnp.maximum(h, 0.0)
    out_ref[...] = (lax.dot_general(h, w2_ref[...], (((1,), (0,)), ((), ())),
                                    preferred_element_type=jnp.float32)
                    + b2_ref[...])


def _mlp(feats_flat, wn, x_skip, w1a, w1b, b1r, W2, b2r):
    n_f, d_skip = x_skip.shape
    d_in = w1a.shape[0]
    d_hid = w1a.shape[1]
    d_out = W2.shape[1]
    nb = n_f // MB
    return pl.pallas_call(
        _mlp_body,
        grid=(nb,),
        in_specs=[
            pl.BlockSpec((MB, d_in), lambda i: (i, 0)),
            pl.BlockSpec((MB, d_in), lambda i: (i + nb, 0)),
            pl.BlockSpec((MB, d_in), lambda i: (i + 2 * nb, 0)),
            pl.BlockSpec((MB, 3), lambda i: (i, 0)),
            pl.BlockSpec((MB, d_skip), lambda i: (i, 0)),
            pl.BlockSpec((d_in, d_hid), lambda i: (0, 0)),
            pl.BlockSpec((d_skip, d_hid), lambda i: (0, 0)),
            pl.BlockSpec((1, d_hid), lambda i: (0, 0)),
            pl.BlockSpec((d_hid, d_out), lambda i: (0, 0)),
            pl.BlockSpec((1, d_out), lambda i: (0, 0)),
        ],
        out_specs=pl.BlockSpec((MB, d_out), lambda i: (i, 0)),
        out_shape=jax.ShapeDtypeStruct((n_f, d_out), jnp.float32),
    )(feats_flat, feats_flat, feats_flat, wn, x_skip, w1a, w1b, b1r, W2, b2r)


def kernel(x, pos, batch, x_skip, pos_skip, batch_skip, W1, b1, W2, b2):
    n_c, d_in = x.shape
    n_f, d_skip = x_skip.shape
    d_hid = W1.shape[1]
    d_out = W2.shape[1]

    p_sq_row = jnp.sum(pos * pos, axis=-1)[None, :]
    q_sq = jnp.sum(pos_skip * pos_skip, axis=-1)
    qaug = jnp.concatenate([pos_skip, q_sq[:, None]], 1)
    idx, wn = _knn(pos.T, p_sq_row, qaug)
    feats_flat = _make_gather(n_c, n_f, d_in)(x, idx.T.reshape(3 * n_f))
    out = _mlp(feats_flat, wn, x_skip, W1[:d_in], W1[d_in:],
               b1.reshape(1, d_hid), W2, b2.reshape(1, d_out))
    return (out, pos_skip, batch_skip)


# speedup vs baseline: 9.3785x; 1.2132x over previous
import functools

import jax
import jax.numpy as jnp
from jax import lax
from jax.experimental import pallas as pl
from jax.experimental.pallas import tpu as pltpu
from jax.experimental.pallas import tpu_sc as plsc

QB = 512
MB = 512
GCH = 128


def _rne_bf16(v):
    u = lax.bitcast_convert_type(v, jnp.uint32)
    r = (u + jnp.uint32(0x7FFF) + ((u >> jnp.uint32(16)) & jnp.uint32(1))) \
        & jnp.uint32(0xFFFF0000)
    return lax.bitcast_convert_type(r, jnp.float32)


def _knn_body(pos_t_ref, p_sq_ref, qaug_ref, idx_ref, wn_ref):
    n_c = pos_t_ref.shape[1]
    qb = qaug_ref.shape[0]
    qp2 = None
    for c in range(3):
        pc = _rne_bf16(pos_t_ref[c:c + 1, :])
        qc2 = _rne_bf16(qaug_ref[:, c:c + 1]) * (-2.0)
        qp2 = qc2 * pc if qp2 is None else qp2 + qc2 * pc
    q_sq = qaug_ref[:, 3:4]
    d2 = (q_sq + p_sq_ref[...]) + qp2

    iota = lax.broadcasted_iota(jnp.int32, (qb, n_c), 1)
    inf = jnp.float32(jnp.inf)
    d = d2
    ws, ims = [], []
    wsum = None
    for t in range(3):
        m = jnp.min(d, axis=1, keepdims=True)
        im = jnp.min(jnp.where(d == m, iota, n_c), axis=1, keepdims=True)
        w = 1.0 / jnp.maximum(m, 1e-16)
        ws.append(w)
        ims.append(im)
        wsum = w if wsum is None else wsum + w
        if t < 2:
            d = jnp.where(iota == im, inf, d)
    for t in range(3):
        idx_ref[:, t:t + 1] = ims[t]
        wn_ref[:, t:t + 1] = ws[t] / wsum


def _knn(pos_t_r, p_sq_row, qaug):
    n_c = pos_t_r.shape[1]
    n_f = qaug.shape[0]
    return pl.pallas_call(
        _knn_body,
        grid=(n_f // QB,),
        in_specs=[
            pl.BlockSpec((3, n_c), lambda i: (0, 0)),
            pl.BlockSpec((1, n_c), lambda i: (0, 0)),
            pl.BlockSpec((QB, 4), lambda i: (i, 0)),
        ],
        out_specs=[
            pl.BlockSpec((QB, 3), lambda i: (i, 0)),
            pl.BlockSpec((QB, 3), lambda i: (i, 0)),
        ],
        out_shape=[
            jax.ShapeDtypeStruct((n_f, 3), jnp.int32),
            jax.ShapeDtypeStruct((n_f, 3), jnp.float32),
        ],
    )(pos_t_r, p_sq_row, qaug)


@functools.cache
def _make_gather(n_c, n_f, d_in):
    info = plsc.get_sparse_core_info()
    n_cores = info.num_cores
    nw = n_cores * info.num_subcores
    bpw = n_f // nw
    nch = bpw // GCH
    mesh = plsc.VectorSubcoreMesh(core_axis_name="c", subcore_axis_name="s")

    @functools.partial(
        pl.kernel,
        out_type=jax.ShapeDtypeStruct((3 * n_f, d_in), jnp.float32),
        mesh=mesh,
        scratch_types=[
            pltpu.VMEM((3, GCH), jnp.int32),
            pltpu.VMEM((3, GCH, d_in), jnp.float32),
            pltpu.SemaphoreType.DMA,
            pltpu.SemaphoreType.DMA,
            pltpu.SemaphoreType.DMA,
        ],
    )
    def gather(x_hbm, idx_hbm, out_hbm, idx_v, rows_v, s0, s1, s2):
        wid = lax.axis_index("s") * n_cores + lax.axis_index("c")
        base = wid * bpw
        sems = (s0, s1, s2)

        def chunk(i, carry):
            qoff = base + i * GCH
            for j in range(3):
                pltpu.sync_copy(idx_hbm.at[pl.ds(j * n_f + qoff, GCH)],
                                idx_v.at[j])
            copies = [pltpu.async_copy(x_hbm.at[idx_v.at[j]], rows_v.at[j],
                                       sems[j]) for j in range(3)]
            for j in range(3):
                copies[j].wait()
                pltpu.sync_copy(rows_v.at[j],
                                out_hbm.at[pl.ds(j * n_f + qoff, GCH)])
            return carry

        lax.fori_loop(0, nch, chunk, 0)

    return gather


def _mlp_body(f0_ref, f1_ref, f2_ref, wn_ref, xs_ref, w1a_ref, w1b_ref,
              b1_ref, w2_ref, b2_ref, out_ref):
    y = None
    for t, f_ref in enumerate((f0_ref, f1_ref, f2_ref)):
        ft = f_ref[...]
        wt = wn_ref[:, t:t + 1]
        y = wt * ft if y is None else y + wt * ft
    h = (lax.dot_general(y, w1a_ref[...], (((1,), (0,)), ((), ())),
                         preferred_element_type=jnp.float32)
         + lax.dot_general(xs_ref[...], w1b_ref[...], (((1,), (0,)), ((), ())),
                           preferred_element_type=jnp.float32)
         + b1_ref[...])
    h = jnp.maximum(h, 0.0)
    out_ref[...] = (lax.dot_general(h, w2_ref[...], (((1,), (0,)), ((), ())),
                                    preferred_element_type=jnp.float32)
                    + b2_ref[...])


def _mlp(feats_flat, wn, x_skip, w1a, w1b, b1r, W2, b2r):
    n_f, d_skip = x_skip.shape
    d_in = w1a.shape[0]
    d_hid = w1a.shape[1]
    d_out = W2.shape[1]
    nb = n_f // MB
    return pl.pallas_call(
        _mlp_body,
        grid=(nb,),
        in_specs=[
            pl.BlockSpec((MB, d_in), lambda i: (i, 0)),
            pl.BlockSpec((MB, d_in), lambda i: (i + nb, 0)),
            pl.BlockSpec((MB, d_in), lambda i: (i + 2 * nb, 0)),
            pl.BlockSpec((MB, 3), lambda i: (i, 0)),
            pl.BlockSpec((MB, d_skip), lambda i: (i, 0)),
            pl.BlockSpec((d_in, d_hid), lambda i: (0, 0)),
            pl.BlockSpec((d_skip, d_hid), lambda i: (0, 0)),
            pl.BlockSpec((1, d_hid), lambda i: (0, 0)),
            pl.BlockSpec((d_hid, d_out), lambda i: (0, 0)),
            pl.BlockSpec((1, d_out), lambda i: (0, 0)),
        ],
        out_specs=pl.BlockSpec((MB, d_out), lambda i: (i, 0)),
        out_shape=jax.ShapeDtypeStruct((n_f, d_out), jnp.float32),
    )(feats_flat, feats_flat, feats_flat, wn, x_skip, w1a, w1b, b1r, W2, b2r)


def kernel(x, pos, batch, x_skip, pos_skip, batch_skip, W1, b1, W2, b2):
    n_c, d_in = x.shape
    n_f, d_skip = x_skip.shape
    d_hid = W1.shape[1]
    d_out = W2.shape[1]

    p_sq_row = jnp.sum(pos * pos, axis=-1)[None, :]
    q_sq = jnp.sum(pos_skip * pos_skip, axis=-1)
    qaug = jnp.concatenate([pos_skip, q_sq[:, None]], 1)
    idx, wn = _knn(pos.T, p_sq_row, qaug)
    feats_flat = _make_gather(n_c, n_f, d_in)(x, idx.T.reshape(3 * n_f))
    out = _mlp(feats_flat, wn, x_skip, W1[:d_in], W1[d_in:],
               b1.reshape(1, d_hid), W2, b2.reshape(1, d_out))
    return (out, pos_skip, batch_skip)
